# baseline (speedup 1.0000x reference)
import jax
import jax.numpy as jnp
from jax import lax
from jax.experimental import pallas as pl
from jax.experimental.pallas import tpu as pltpu
from jax.experimental.pallas import tpu_sc as plsc

Z = 384
B = 64
K = 8448
N = 16896
EXT_ROWS = 20
CW = K + 4 * Z
ZSLOT = CW
NCHUNK = Z // 16

NC = 2
NS = 16
ROWS_PER_W = B // (NC * NS)


def _sc_body(bits_hbm, g1_hbm, g2_hbm, p0i_hbm, out_hbm,
             cw_v, g1_v, g2_v, p0i_v, m_v, mt_v, ext_v):
    wid = lax.axis_index("s") * NC + lax.axis_index("c")

    pltpu.sync_copy(g1_hbm, g1_v)
    pltpu.sync_copy(g2_hbm, g2_v)
    pltpu.sync_copy(p0i_hbm, p0i_v)
    cw_v[pl.ds(CW, 16)] = jnp.zeros((16,), jnp.float32)

    for k in range(ROWS_PER_W):
        b = wid * ROWS_PER_W + k

        pltpu.sync_copy(bits_hbm.at[pl.ds(b * K, K)], cw_v.at[pl.ds(0, K)])

        def stage1(j, carry):
            off = j * 16
            ms = []
            for r in range(4):
                acc = None
                for e in range(22):
                    idx = g1_v[pl.ds((r * 22 + e) * Z + off, 16)]
                    g = plsc.load_gather(cw_v, [idx])
                    acc = g if acc is None else acc + g
                mr = lax.rem(acc, 2.0)
                ms.append(mr)
                if r > 0:
                    m_v[pl.ds(r * Z + off, 16)] = mr
            mt_v[pl.ds(off, 16)] = lax.rem(ms[0] + ms[1] + ms[2] + ms[3], 2.0)
            return carry

        lax.fori_loop(0, NCHUNK, stage1, 0)

        def stage2(j, carry):
            off = j * 16
            pidx = p0i_v[pl.ds(off, 16)]
            p0 = plsc.load_gather(mt_v, [pidx])
            m1 = m_v[pl.ds(1 * Z + off, 16)]
            m2 = m_v[pl.ds(2 * Z + off, 16)]
            m3 = m_v[pl.ds(3 * Z + off, 16)]
            p1 = lax.rem(m1 + m2 + m3, 2.0)
            p3 = lax.rem(m3 + p0, 2.0)
            p2 = lax.rem(m2 + p3, 2.0)
            cw_v[pl.ds(K + 0 * Z + off, 16)] = p0
            cw_v[pl.ds(K + 1 * Z + off, 16)] = p1
            cw_v[pl.ds(K + 2 * Z + off, 16)] = p2
            cw_v[pl.ds(K + 3 * Z + off, 16)] = p3
            return carry

        lax.fori_loop(0, NCHUNK, stage2, 0)

        def stage3(j, carry):
            off = j * 16
            for r in range(EXT_ROWS):
                acc = None
                for e in range(4):
                    idx = g2_v[pl.ds((r * 4 + e) * Z + off, 16)]
                    g = plsc.load_gather(cw_v, [idx])
                    acc = g if acc is None else acc + g
                ext_v[pl.ds(r * Z + off, 16)] = lax.rem(acc, 2.0)
            return carry

        lax.fori_loop(0, NCHUNK, stage3, 0)

        ob = b * N
        pltpu.sync_copy(cw_v.at[pl.ds(2 * Z, K - 2 * Z)],
                        out_hbm.at[pl.ds(ob, K - 2 * Z)])
        pltpu.sync_copy(cw_v.at[pl.ds(K, 4 * Z)],
                        out_hbm.at[pl.ds(ob + K - 2 * Z, 4 * Z)])
        pltpu.sync_copy(ext_v,
                        out_hbm.at[pl.ds(ob + K + 2 * Z, EXT_ROWS * Z)])


def kernel(inputs, A_r, A_c, A_s, C_r, C_c, C_s):
    bits = inputs.astype(jnp.float32).reshape(B * K)
    ar = jnp.asarray(A_r, jnp.int32)
    ac = jnp.asarray(A_c, jnp.int32)
    ash = jnp.asarray(A_s, jnp.int32)
    cc = jnp.asarray(C_c, jnp.int32)
    cs = jnp.asarray(C_s, jnp.int32)
    del C_r
    na = ar.shape[0]

    iota = jnp.arange(Z, dtype=jnp.int32)
    g1_rows = ac[:, None] * Z + (iota[None, :] + ash[:, None]) % Z
    perm = jnp.argsort(ar, stable=True)
    r_sorted = ar[perm]
    first = jnp.searchsorted(r_sorted, jnp.arange(4, dtype=jnp.int32))
    rank = jnp.arange(na, dtype=jnp.int32) - first[r_sorted]
    slots = r_sorted * 22 + rank
    g1 = jnp.full((4 * 22, Z), ZSLOT, jnp.int32).at[slots].set(g1_rows[perm])
    g1 = g1.reshape(-1)
    g2 = (cc[:4 * EXT_ROWS, None] * Z
          + (iota[None, :] + cs[:4 * EXT_ROWS, None]) % Z).reshape(-1)
    p0i = (iota + Z - 1) % Z

    mesh = plsc.VectorSubcoreMesh(core_axis_name="c", subcore_axis_name="s")
    out = pl.kernel(
        _sc_body,
        out_type=jax.ShapeDtypeStruct((B * N,), jnp.float32),
        mesh=mesh,
        compiler_params=pltpu.CompilerParams(needs_layout_passes=False),
        scratch_types=[
            pltpu.VMEM((CW + 16,), jnp.float32),
            pltpu.VMEM((4 * 22 * Z,), jnp.int32),
            pltpu.VMEM((4 * EXT_ROWS * Z,), jnp.int32),
            pltpu.VMEM((Z,), jnp.int32),
            pltpu.VMEM((4 * Z,), jnp.float32),
            pltpu.VMEM((Z,), jnp.float32),
            pltpu.VMEM((EXT_ROWS * Z,), jnp.float32),
        ],
    )(bits, g1, g2, p0i)
    return out.reshape(B, N)
